# Initial kernel scaffold; baseline (speedup 1.0000x reference)
import jax
import jax.numpy as jnp
from jax.experimental import pallas as pl


def kernel(x, w_t, b, bn_gamma, bn_beta, bn_mean, bn_var):
    raise NotImplementedError("write your pallas kernel here")


# speedup vs baseline: 2.3885x; 2.3885x over previous
import jax
import jax.numpy as jnp
from jax.experimental import pallas as pl
from jax.experimental.pallas import tpu as pltpu


def _round_up(x, m):
    return (x + m - 1) // m * m


def _fused_rowblock_kernel(x_ref, w_ref, s_ref, t_ref, o_ref):
    xb = x_ref[...].astype(jnp.bfloat16)
    acc = jnp.dot(xb, w_ref[...], preferred_element_type=jnp.float32)
    y = acc * s_ref[...] + t_ref[...]
    o_ref[...] = jnp.maximum(y, 0.0).astype(o_ref.dtype)


def _fused_linear_bn_relu(x2d, w_bf16, scale, shift, *, bm=512):
    M, Din = x2d.shape
    Dout = w_bf16.shape[1]

    bm = min(bm, _round_up(M, 8))
    Mp = _round_up(M, bm)
    if Mp != M:
        x2d = jnp.pad(x2d, ((0, Mp - M), (0, 0)))

    s2 = scale.reshape(1, Dout).astype(jnp.float32)
    t2 = shift.reshape(1, Dout).astype(jnp.float32)

    flops = 2 * Mp * Din * Dout
    bytes_accessed = Mp * Din * 4 + Din * Dout * 2 + Mp * Dout * 4 + 2 * Dout * 4
    cost = pl.CostEstimate(flops=flops, transcendentals=0,
                           bytes_accessed=bytes_accessed)

    out = pl.pallas_call(
        _fused_rowblock_kernel,
        grid=(Mp // bm,),
        out_shape=jax.ShapeDtypeStruct((Mp, Dout), x2d.dtype),
        in_specs=[
            pl.BlockSpec((bm, Din), lambda i: (i, 0)),
            pl.BlockSpec((Din, Dout), lambda i: (0, 0)),
            pl.BlockSpec((1, Dout), lambda i: (0, 0)),
            pl.BlockSpec((1, Dout), lambda i: (0, 0)),
        ],
        out_specs=pl.BlockSpec((bm, Dout), lambda i: (i, 0)),
        compiler_params=pltpu.CompilerParams(
            dimension_semantics=("parallel",),
            vmem_limit_bytes=100 * 1024 * 1024,
        ),
        cost_estimate=cost,
    )(x2d, w_bf16, s2, t2)

    return out[:M] if Mp != M else out


def kernel(x, w_t, b, bn_gamma, bn_beta, bn_mean, bn_var):
    eps = 1e-5
    s = bn_gamma * jax.lax.rsqrt(bn_var + eps)
    t = (b - bn_mean) * s + bn_beta

    w_bf16 = w_t.astype(jnp.bfloat16)

    if x.ndim == 3:
        N, K, Din = x.shape
        y = _fused_linear_bn_relu(x.reshape(N * K, Din), w_bf16, s, t)
        return y.reshape(N, K, -1)
    return _fused_linear_bn_relu(x, w_bf16, s, t)


# speedup vs baseline: 2.6593x; 1.1134x over previous
import jax
import jax.numpy as jnp
from jax.experimental import pallas as pl
from jax.experimental.pallas import tpu as pltpu


def _round_up(x, m):
    return (x + m - 1) // m * m


def _fused_rowblock_kernel(x_ref, w_ref, s_ref, t_ref, o_ref, wb_ref):
    j = pl.program_id(1)

    @pl.when(j == 0)
    def _():
        wb_ref[...] = w_ref[...].astype(jnp.bfloat16)

    xb = x_ref[...].astype(jnp.bfloat16)
    acc = jnp.dot(xb, wb_ref[...], preferred_element_type=jnp.float32)
    y = acc * s_ref[...] + t_ref[...]
    o_ref[...] = jnp.maximum(y, 0.0).astype(o_ref.dtype)


_NUM_CORES = 2


def _fused_linear_bn_relu(x2d, w_t, scale, shift, *, bm=512):
    M, Din = x2d.shape
    Dout = w_t.shape[1]

    bm = min(bm, _round_up(M, 8))
    Mp = _round_up(M, _NUM_CORES * bm)
    if Mp != M:
        x2d = jnp.pad(x2d, ((0, Mp - M), (0, 0)))
    nsteps = Mp // bm // _NUM_CORES

    s2 = scale.reshape(1, Dout).astype(jnp.float32)
    t2 = shift.reshape(1, Dout).astype(jnp.float32)

    flops = 2 * Mp * Din * Dout
    bytes_accessed = Mp * Din * 4 + Din * Dout * 4 + Mp * Dout * 4 + 2 * Dout * 4
    cost = pl.CostEstimate(flops=flops, transcendentals=0,
                           bytes_accessed=bytes_accessed)

    out = pl.pallas_call(
        _fused_rowblock_kernel,
        grid=(_NUM_CORES, nsteps),
        out_shape=jax.ShapeDtypeStruct((Mp, Dout), x2d.dtype),
        in_specs=[
            pl.BlockSpec((bm, Din), lambda c, j: (c * nsteps + j, 0)),
            pl.BlockSpec((Din, Dout), lambda c, j: (0, 0)),
            pl.BlockSpec((1, Dout), lambda c, j: (0, 0)),
            pl.BlockSpec((1, Dout), lambda c, j: (0, 0)),
        ],
        out_specs=pl.BlockSpec((bm, Dout), lambda c, j: (c * nsteps + j, 0)),
        scratch_shapes=[pltpu.VMEM((Din, Dout), jnp.bfloat16)],
        compiler_params=pltpu.CompilerParams(
            dimension_semantics=("parallel", "arbitrary"),
            vmem_limit_bytes=100 * 1024 * 1024,
        ),
        cost_estimate=cost,
    )(x2d, w_t, s2, t2)

    return out[:M] if Mp != M else out


def kernel(x, w_t, b, bn_gamma, bn_beta, bn_mean, bn_var):
    eps = 1e-5
    s = bn_gamma * jax.lax.rsqrt(bn_var + eps)
    t = (b - bn_mean) * s + bn_beta

    if x.ndim == 3:
        N, K, Din = x.shape
        y = _fused_linear_bn_relu(x.reshape(N * K, Din), w_t, s, t)
        return y.reshape(N, K, -1)
    return _fused_linear_bn_relu(x, w_t, s, t)
